# baseline (speedup 1.0000x reference)
import jax
import jax.numpy as jnp
from jax.experimental import pallas as pl
from jax.experimental.pallas import tpu as pltpu

_H, _CH = 8, 64
_CD = _H * _CH
_ALPHA = 0.2
_NEG = -1152921504606846976.0
_LOG2E = 1.4426950408889634

_TM = 512
_TI = 256
_RS = 32


def _proj_kernel(x_ref, wt_ref, b_ref, a2_ref, h_ref, e_ref):
    hp = jnp.dot(x_ref[...], wt_ref[...], preferred_element_type=jnp.float32)
    hp = hp + b_ref[...]
    h_ref[...] = hp
    e_ref[...] = jnp.dot(hp, a2_ref[...], preferred_element_type=jnp.float32)


def _attn_kernel(er_ref, ect_ref, adj_ref, h_ref, out_ref, atten_ref,
                 pen_ref):
    pen_ref[...] = jnp.where(adj_ref[0] == 1, 0.0, _NEG)
    ti = adj_ref.shape[1]
    for hh in range(_H):
        ec = ect_ref[0, hh:hh + 1, :]
        for r in range(ti // _RS):
            rsl = pl.ds(r * _RS, _RS)
            er = er_ref[0, rsl, hh:hh + 1]
            logit = er + ec
            leaky = jnp.maximum(logit, _ALPHA * logit)
            masked = leaky + pen_ref[rsl, :]
            m = jnp.max(masked, axis=1, keepdims=True)
            p = jnp.exp(masked - m)
            probs = p / jnp.sum(p, axis=1, keepdims=True)
            atten_ref[0, hh, rsl, :] = probs
        hv = h_ref[0, :, hh * _CH:(hh + 1) * _CH]
        out_ref[0, :, hh * _CH:(hh + 1) * _CH] = jnp.dot(
            atten_ref[0, hh, :, :], hv, preferred_element_type=jnp.float32)


def kernel(node_feats, adj_matrix, W, b, a):
    B, N, C_IN = node_feats.shape
    x = node_feats.reshape(B * N, C_IN)
    wt = W.T
    a_src = a[:, :_CH].reshape(-1, 1)
    a_dst = a[:, _CH:].reshape(-1, 1)
    eye = jnp.repeat(jnp.eye(_H, dtype=jnp.float32), _CH, axis=0)
    a2 = jnp.concatenate([eye * a_src, eye * a_dst], axis=1)
    b2 = b.reshape(1, _CD)

    h_flat, e = pl.pallas_call(
        _proj_kernel,
        grid=(B * N // _TM,),
        in_specs=[
            pl.BlockSpec((_TM, C_IN), lambda i: (i, 0)),
            pl.BlockSpec((C_IN, _CD), lambda i: (0, 0)),
            pl.BlockSpec((1, _CD), lambda i: (0, 0)),
            pl.BlockSpec((C_IN, 2 * _H), lambda i: (0, 0)),
        ],
        out_specs=[
            pl.BlockSpec((_TM, _CD), lambda i: (i, 0)),
            pl.BlockSpec((_TM, 2 * _H), lambda i: (i, 0)),
        ],
        out_shape=[
            jax.ShapeDtypeStruct((B * N, _CD), jnp.float32),
            jax.ShapeDtypeStruct((B * N, 2 * _H), jnp.float32),
        ],
    )(x, wt, b2, a2)

    h = h_flat.reshape(B, N, _CD)
    e = e.reshape(B, N, 2 * _H)
    er = e[:, :, :_H]
    ect = jnp.transpose(e[:, :, _H:], (0, 2, 1))

    out, atten = pl.pallas_call(
        _attn_kernel,
        grid=(B, N // _TI),
        in_specs=[
            pl.BlockSpec((1, _TI, _H), lambda bb, i: (bb, i, 0)),
            pl.BlockSpec((1, _H, N), lambda bb, i: (bb, 0, 0)),
            pl.BlockSpec((1, _TI, N), lambda bb, i: (bb, i, 0)),
            pl.BlockSpec((1, N, _CD), lambda bb, i: (bb, 0, 0)),
        ],
        out_specs=[
            pl.BlockSpec((1, _TI, _CD), lambda bb, i: (bb, i, 0)),
            pl.BlockSpec((1, _H, _TI, N), lambda bb, i: (bb, 0, i, 0)),
        ],
        out_shape=[
            jax.ShapeDtypeStruct((B, N, _CD), jnp.float32),
            jax.ShapeDtypeStruct((B, _H, N, N), jnp.float32),
        ],
        scratch_shapes=[pltpu.VMEM((_TI, N), jnp.float32)],
    )(er, ect, adj_matrix, h)

    return (out, atten)


# speedup vs baseline: 1.4658x; 1.4658x over previous
import jax
import jax.numpy as jnp
from jax.experimental import pallas as pl
from jax.experimental.pallas import tpu as pltpu

_H, _CH = 8, 64
_CD = _H * _CH
_ALPHA = 0.2
_NEG = -1152921504606846976.0
_LOG2E = 1.4426950408889634

_TM = 512
_TI = 256
_RS = 64


def _proj_kernel(x_ref, wt_ref, b_ref, a2_ref, h_ref, e_ref):
    hp = jnp.dot(x_ref[...], wt_ref[...], preferred_element_type=jnp.float32)
    hp = hp + b_ref[...]
    h_ref[...] = hp
    e_ref[...] = jnp.dot(hp, a2_ref[...], preferred_element_type=jnp.float32)


def _attn_kernel(er_ref, ect_ref, adj_ref, h_ref, out_ref, atten_ref,
                 pen_ref):
    pen_ref[...] = jnp.where(adj_ref[0] == 1, 0.0, _NEG)
    ti = adj_ref.shape[1]
    for hh in range(_H):
        ec = ect_ref[0, hh:hh + 1, :]
        for r in range(ti // _RS):
            rsl = pl.ds(r * _RS, _RS)
            er = er_ref[0, rsl, hh:hh + 1]
            logit = er + ec
            leaky = jnp.maximum(logit, _ALPHA * logit)
            masked = leaky + pen_ref[rsl, :]
            m = jnp.max(masked, axis=1, keepdims=True)
            p = jnp.exp(masked - m)
            probs = p / jnp.sum(p, axis=1, keepdims=True)
            atten_ref[0, hh, rsl, :] = probs
        hv = h_ref[0, :, hh * _CH:(hh + 1) * _CH]
        out_ref[0, :, hh * _CH:(hh + 1) * _CH] = jnp.dot(
            atten_ref[0, hh, :, :], hv, preferred_element_type=jnp.float32)


def kernel(node_feats, adj_matrix, W, b, a):
    B, N, C_IN = node_feats.shape
    x = node_feats.reshape(B * N, C_IN)
    wt = W.T
    a_src = a[:, :_CH].reshape(-1, 1)
    a_dst = a[:, _CH:].reshape(-1, 1)
    eye = jnp.repeat(jnp.eye(_H, dtype=jnp.float32), _CH, axis=0)
    a2 = jnp.concatenate([eye * a_src, eye * a_dst], axis=1)
    b2 = b.reshape(1, _CD)

    h_flat, e = pl.pallas_call(
        _proj_kernel,
        grid=(B * N // _TM,),
        in_specs=[
            pl.BlockSpec((_TM, C_IN), lambda i: (i, 0)),
            pl.BlockSpec((C_IN, _CD), lambda i: (0, 0)),
            pl.BlockSpec((1, _CD), lambda i: (0, 0)),
            pl.BlockSpec((C_IN, 2 * _H), lambda i: (0, 0)),
        ],
        out_specs=[
            pl.BlockSpec((_TM, _CD), lambda i: (i, 0)),
            pl.BlockSpec((_TM, 2 * _H), lambda i: (i, 0)),
        ],
        out_shape=[
            jax.ShapeDtypeStruct((B * N, _CD), jnp.float32),
            jax.ShapeDtypeStruct((B * N, 2 * _H), jnp.float32),
        ],
    )(x, wt, b2, a2)

    h = h_flat.reshape(B, N, _CD)
    e = e.reshape(B, N, 2 * _H)
    er = e[:, :, :_H]
    ect = jnp.transpose(e[:, :, _H:], (0, 2, 1))

    out, atten = pl.pallas_call(
        _attn_kernel,
        grid=(B, N // _TI),
        in_specs=[
            pl.BlockSpec((1, _TI, _H), lambda bb, i: (bb, i, 0)),
            pl.BlockSpec((1, _H, N), lambda bb, i: (bb, 0, 0)),
            pl.BlockSpec((1, _TI, N), lambda bb, i: (bb, i, 0)),
            pl.BlockSpec((1, N, _CD), lambda bb, i: (bb, 0, 0)),
        ],
        out_specs=[
            pl.BlockSpec((1, _TI, _CD), lambda bb, i: (bb, i, 0)),
            pl.BlockSpec((1, _H, _TI, N), lambda bb, i: (bb, 0, i, 0)),
        ],
        out_shape=[
            jax.ShapeDtypeStruct((B, N, _CD), jnp.float32),
            jax.ShapeDtypeStruct((B, _H, N, N), jnp.float32),
        ],
        scratch_shapes=[pltpu.VMEM((_TI, N), jnp.float32)],
    )(er, ect, adj_matrix, h)

    return (out, atten)


# speedup vs baseline: 1.8809x; 1.2832x over previous
import jax
import jax.numpy as jnp
from jax.experimental import pallas as pl
from jax.experimental.pallas import tpu as pltpu

_H, _CH = 8, 64
_CD = _H * _CH
_ALPHA = 0.2
_NEG = -1152921504606846976.0
_LOG2E = 1.4426950408889634

_TM = 512
_TI = 256
_RS = 128


def _proj_kernel(x_ref, wt_ref, b_ref, a2_ref, h_ref, e_ref):
    hp = jnp.dot(x_ref[...], wt_ref[...], preferred_element_type=jnp.float32)
    hp = hp + b_ref[...]
    h_ref[...] = hp
    e_ref[...] = jnp.dot(hp, a2_ref[...], preferred_element_type=jnp.float32)


def _attn_kernel(er_ref, ect_ref, adj_ref, h_ref, out_ref, atten_ref,
                 pen_ref):
    pen_ref[...] = jnp.where(adj_ref[0] == 1, 0.0, _NEG)
    ti = adj_ref.shape[1]
    for hh in range(_H):
        ec = ect_ref[0, hh:hh + 1, :]
        for r in range(ti // _RS):
            rsl = pl.ds(r * _RS, _RS)
            er = er_ref[0, rsl, hh:hh + 1]
            logit = er + ec
            leaky = jnp.maximum(logit, _ALPHA * logit)
            masked = leaky + pen_ref[rsl, :]
            m = jnp.max(masked, axis=1, keepdims=True)
            p = jnp.exp(masked - m)
            probs = p / jnp.sum(p, axis=1, keepdims=True)
            atten_ref[0, hh, rsl, :] = probs
        hv = h_ref[0, :, hh * _CH:(hh + 1) * _CH]
        out_ref[0, :, hh * _CH:(hh + 1) * _CH] = jnp.dot(
            atten_ref[0, hh, :, :], hv, preferred_element_type=jnp.float32)


def kernel(node_feats, adj_matrix, W, b, a):
    B, N, C_IN = node_feats.shape
    x = node_feats.reshape(B * N, C_IN)
    wt = W.T
    a_src = a[:, :_CH].reshape(-1, 1)
    a_dst = a[:, _CH:].reshape(-1, 1)
    eye = jnp.repeat(jnp.eye(_H, dtype=jnp.float32), _CH, axis=0)
    a2 = jnp.concatenate([eye * a_src, eye * a_dst], axis=1)
    b2 = b.reshape(1, _CD)

    h_flat, e = pl.pallas_call(
        _proj_kernel,
        grid=(B * N // _TM,),
        in_specs=[
            pl.BlockSpec((_TM, C_IN), lambda i: (i, 0)),
            pl.BlockSpec((C_IN, _CD), lambda i: (0, 0)),
            pl.BlockSpec((1, _CD), lambda i: (0, 0)),
            pl.BlockSpec((C_IN, 2 * _H), lambda i: (0, 0)),
        ],
        out_specs=[
            pl.BlockSpec((_TM, _CD), lambda i: (i, 0)),
            pl.BlockSpec((_TM, 2 * _H), lambda i: (i, 0)),
        ],
        out_shape=[
            jax.ShapeDtypeStruct((B * N, _CD), jnp.float32),
            jax.ShapeDtypeStruct((B * N, 2 * _H), jnp.float32),
        ],
    )(x, wt, b2, a2)

    h = h_flat.reshape(B, N, _CD)
    e = e.reshape(B, N, 2 * _H)
    er = e[:, :, :_H]
    ect = jnp.transpose(e[:, :, _H:], (0, 2, 1))

    out, atten = pl.pallas_call(
        _attn_kernel,
        grid=(B, N // _TI),
        in_specs=[
            pl.BlockSpec((1, _TI, _H), lambda bb, i: (bb, i, 0)),
            pl.BlockSpec((1, _H, N), lambda bb, i: (bb, 0, 0)),
            pl.BlockSpec((1, _TI, N), lambda bb, i: (bb, i, 0)),
            pl.BlockSpec((1, N, _CD), lambda bb, i: (bb, 0, 0)),
        ],
        out_specs=[
            pl.BlockSpec((1, _TI, _CD), lambda bb, i: (bb, i, 0)),
            pl.BlockSpec((1, _H, _TI, N), lambda bb, i: (bb, 0, i, 0)),
        ],
        out_shape=[
            jax.ShapeDtypeStruct((B, N, _CD), jnp.float32),
            jax.ShapeDtypeStruct((B, _H, N, N), jnp.float32),
        ],
        scratch_shapes=[pltpu.VMEM((_TI, N), jnp.float32)],
    )(er, ect, adj_matrix, h)

    return (out, atten)


# speedup vs baseline: 2.3317x; 1.2397x over previous
import jax
import jax.numpy as jnp
from jax.experimental import pallas as pl
from jax.experimental.pallas import tpu as pltpu

_H, _CH = 8, 64
_CD = _H * _CH
_ALPHA = 0.2
_NEG = -1152921504606846976.0

_TI = 256


def _gat_kernel(x_ref, wt_ref, b_ref, a2_ref, adj_ref, out_ref, atten_ref,
                h_s, er_s, ect_s, pen_ref):
    i = pl.program_id(1)

    @pl.when(i == 0)
    def _project():
        hp = jnp.dot(x_ref[0], wt_ref[...],
                     preferred_element_type=jnp.float32) + b_ref[...]
        h_s[...] = hp
        e = jnp.dot(hp, a2_ref[...], preferred_element_type=jnp.float32)
        er_s[...] = e[:, :_H]
        ect_s[...] = jnp.transpose(e[:, _H:], (1, 0))

    pen_ref[...] = jnp.where(adj_ref[0] == 1, 0.0, _NEG)
    pen = pen_ref[...]
    for hh in range(_H):
        er = er_s[pl.ds(i * _TI, _TI), hh:hh + 1]
        ec = ect_s[hh:hh + 1, :]
        logit = er + ec
        leaky = jnp.maximum(logit, _ALPHA * logit)
        masked = leaky + pen
        m = jnp.max(masked, axis=1, keepdims=True)
        p = jnp.exp(masked - m)
        probs = p / jnp.sum(p, axis=1, keepdims=True)
        atten_ref[0, hh, :, :] = probs
        hv = h_s[:, hh * _CH:(hh + 1) * _CH]
        out_ref[0, :, hh * _CH:(hh + 1) * _CH] = jnp.dot(
            probs, hv, preferred_element_type=jnp.float32)


def kernel(node_feats, adj_matrix, W, b, a):
    B, N, C_IN = node_feats.shape
    wt = W.T
    a_src = a[:, :_CH].reshape(-1, 1)
    a_dst = a[:, _CH:].reshape(-1, 1)
    eye = jnp.repeat(jnp.eye(_H, dtype=jnp.float32), _CH, axis=0)
    a2 = jnp.concatenate([eye * a_src, eye * a_dst], axis=1)
    b2 = b.reshape(1, _CD)

    out, atten = pl.pallas_call(
        _gat_kernel,
        grid=(B, N // _TI),
        in_specs=[
            pl.BlockSpec((1, N, C_IN), lambda bb, i: (bb, 0, 0)),
            pl.BlockSpec((C_IN, _CD), lambda bb, i: (0, 0)),
            pl.BlockSpec((1, _CD), lambda bb, i: (0, 0)),
            pl.BlockSpec((C_IN, 2 * _H), lambda bb, i: (0, 0)),
            pl.BlockSpec((1, _TI, N), lambda bb, i: (bb, i, 0)),
        ],
        out_specs=[
            pl.BlockSpec((1, _TI, _CD), lambda bb, i: (bb, i, 0)),
            pl.BlockSpec((1, _H, _TI, N), lambda bb, i: (bb, 0, i, 0)),
        ],
        out_shape=[
            jax.ShapeDtypeStruct((B, N, _CD), jnp.float32),
            jax.ShapeDtypeStruct((B, _H, N, N), jnp.float32),
        ],
        scratch_shapes=[
            pltpu.VMEM((N, _CD), jnp.float32),
            pltpu.VMEM((N, _H), jnp.float32),
            pltpu.VMEM((_H, N), jnp.float32),
            pltpu.VMEM((_TI, N), jnp.float32),
        ],
    )(node_feats, wt, b2, a2, adj_matrix)

    return (out, atten)
